# baseline (speedup 1.0000x reference)
import functools

import jax
import jax.numpy as jnp
from jax import lax
from jax.experimental import pallas as pl
from jax.experimental.pallas import tpu as pltpu
from jax.experimental.pallas import tpu_sc as plsc

NC = 2
NS = 16
NW = NC * NS

B = 4096
T = 50
D = 128
W = B // NW
NBUF = 5


V = 1000
CHUNK = 64
LAST = V - CHUNK * (NS - 1)


def _gather_body(table_hbm, idx_hbm, out_hbm, idx_v, buf, tab_sh,
                 sg0, sg1, sg2, sg3, sg4):
    sg = (sg0, sg1, sg2, sg3, sg4)
    sid = lax.axis_index("s")
    wid = sid * NC + lax.axis_index("c")
    base = wid * W

    @pl.when(sid < NS - 1)
    def _stage():
        pltpu.sync_copy(table_hbm.at[pl.ds(sid * CHUNK, CHUNK)],
                        tab_sh.at[pl.ds(sid * CHUNK, CHUNK)])

    @pl.when(sid == NS - 1)
    def _stage_last():
        pltpu.sync_copy(table_hbm.at[pl.ds(CHUNK * (NS - 1), LAST)],
                        tab_sh.at[pl.ds(CHUNK * (NS - 1), LAST)])

    @pl.when(sid == 0)
    def _zero_row0():
        for k in range(D // 16):
            buf[0, 0, pl.ds(k * 16, 16)] = jnp.zeros((16,), jnp.float32)
        pltpu.sync_copy(buf.at[0, 0], tab_sh.at[0])

    pltpu.sync_copy(idx_hbm.at[:, pl.ds(base, W)], idx_v)
    plsc.subcore_barrier()

    def start_gather(t, b):
        pltpu.async_copy(tab_sh.at[idx_v.at[t]], buf.at[b], sg[b])

    def wait_gather(t, b):
        pltpu.make_async_copy(
            tab_sh.at[idx_v.at[t]], buf.at[b], sg[b]).wait()

    def scatter(t, b):
        pltpu.sync_copy(buf.at[b], out_hbm.at[t, pl.ds(base, W)])

    for b in range(NBUF):
        start_gather(b, b)

    def outer(i, carry):
        t0 = i * NBUF
        for b in range(NBUF):
            t = t0 + b
            wait_gather(t, b)
            scatter(t, b)
            start_gather(t + NBUF, b)
        return carry

    lax.fori_loop(0, T // NBUF - 1, outer, 0)

    for b in range(NBUF):
        t = T - NBUF + b
        wait_gather(t, b)
        scatter(t, b)


@jax.jit
def _lookup(time_features, table):
    mesh = plsc.VectorSubcoreMesh(core_axis_name="c", subcore_axis_name="s")
    call = functools.partial(
        pl.kernel,
        mesh=mesh,
        out_type=jax.ShapeDtypeStruct((T, B, D), jnp.float32),
        scratch_types=[
            pltpu.VMEM((T, W), jnp.int32),
            pltpu.VMEM((NBUF, W, D), jnp.float32),
            pltpu.VMEM_SHARED((V, D), jnp.float32),
        ] + [pltpu.SemaphoreType.DMA] * NBUF,
    )(_gather_body)
    out_tbd = call(table, time_features.T)
    return jnp.transpose(out_tbd, (1, 0, 2))


def kernel(time_features, table):
    return _lookup(time_features, table)
